# baseline (speedup 1.0000x reference)
import functools
import math

import jax
import jax.numpy as jnp
from jax import lax
from jax.experimental import pallas as pl
from jax.experimental.pallas import tpu as pltpu
from jax.experimental.pallas import tpu_sc as plsc

VOCAB = 100000
EMBED = 128
SCALE = math.sqrt(EMBED)

ROWS = 4096 * 200
NC, NS = 2, 16
NW = NC * NS
RW = ROWS // NW
CH = 64
NCHUNK = RW // CH
NBUF = 8
AHEAD = 4

_mesh = plsc.VectorSubcoreMesh(core_axis_name="c", subcore_axis_name="s")


@functools.partial(
    pl.kernel,
    mesh=_mesh,
    out_type=jax.ShapeDtypeStruct((ROWS, EMBED), jnp.float32),
    scratch_types=(
        [pltpu.VMEM((NCHUNK, CH), jnp.int32)]
        + [pltpu.VMEM((CH, EMBED), jnp.float32)] * NBUF
        + [pltpu.SemaphoreType.DMA] * (2 * NBUF)
    ),
)
def _embed_sc(tok_hbm, tab_hbm, out_hbm, idx_v, *bufs_and_sems):
    rows = bufs_and_sems[:NBUF]
    gsem = bufs_and_sems[NBUF:2 * NBUF]
    osem = bufs_and_sems[2 * NBUF:]
    wid = lax.axis_index("s") * NC + lax.axis_index("c")
    base = wid * RW

    pltpu.sync_copy(tok_hbm.at[wid], idx_v)

    def g_start(c, b):
        pltpu.async_copy(tab_hbm.at[idx_v.at[c]], rows[b], gsem[b])

    def g_wait(c, b):
        pltpu.make_async_copy(tab_hbm.at[idx_v.at[c]], rows[b], gsem[b]).wait()

    def o_start(c, b):
        pltpu.async_copy(rows[b], out_hbm.at[pl.ds(base + c * CH, CH)], osem[b])

    def o_wait(c, b):
        pltpu.make_async_copy(
            rows[b], out_hbm.at[pl.ds(base + c * CH, CH)], osem[b]).wait()

    for c in range(AHEAD):
        g_start(c, c)

    def outer(i, carry):
        j = i * NBUF
        for b in range(NBUF):
            cur = j + b

            @pl.when(cur >= AHEAD)
            def _():
                o_wait(cur - AHEAD, (b + AHEAD) % NBUF)

            @pl.when(cur + AHEAD < NCHUNK)
            def _():
                g_start(cur + AHEAD, (b + AHEAD) % NBUF)

            g_wait(cur, b)

            buf = rows[b]

            @plsc.parallel_loop(0, CH, unroll=4)
            def _(r):
                for c in range(EMBED // 16):
                    buf[r, pl.ds(c * 16, 16)] = buf[r, pl.ds(c * 16, 16)] * SCALE

            o_start(cur, b)
        return carry

    lax.fori_loop(0, NCHUNK // NBUF, outer, 0)
    for c in range(NCHUNK - AHEAD, NCHUNK):
        o_wait(c, c % NBUF)


def kernel(tokens, table):
    tok = tokens.reshape(NW, NCHUNK, CH).astype(jnp.int32)
    out = _embed_sc(tok, table)
    return out.reshape(tokens.shape[0], tokens.shape[1], EMBED)


# speedup vs baseline: 1.0468x; 1.0468x over previous
import functools
import math

import jax
import jax.numpy as jnp
from jax import lax
from jax.experimental import pallas as pl
from jax.experimental.pallas import tpu as pltpu
from jax.experimental.pallas import tpu_sc as plsc

VOCAB = 100000
EMBED = 128
SCALE = math.sqrt(EMBED)

ROWS = 4096 * 200
NC, NS = 2, 16
NW = NC * NS
RW = ROWS // NW
CH = 128
NCHUNK = RW // CH
NBUF = 4
AHEAD = 2

_mesh = plsc.VectorSubcoreMesh(core_axis_name="c", subcore_axis_name="s")


@functools.partial(
    pl.kernel,
    mesh=_mesh,
    out_type=jax.ShapeDtypeStruct((ROWS, EMBED), jnp.float32),
    scratch_types=(
        [pltpu.VMEM((NCHUNK, CH), jnp.int32)]
        + [pltpu.VMEM((CH, EMBED), jnp.float32)] * NBUF
        + [pltpu.VMEM_SHARED((NS, 2, CH, EMBED), jnp.float32)]
        + [pltpu.SemaphoreType.DMA] * (3 * NBUF)
    ),
)
def _embed_sc(tok_hbm, tab_hbm, out_hbm, idx_v, *rest):
    rows = rest[:NBUF]
    shared = rest[NBUF]
    sems = rest[NBUF + 1:]
    gsem = sems[:NBUF]
    s1 = sems[NBUF:2 * NBUF]
    s2 = sems[2 * NBUF:]
    sid = lax.axis_index("s")
    wid = sid * NC + lax.axis_index("c")
    base = wid * RW
    sp = tuple(shared.at[sid, b % 2] for b in range(NBUF))

    pltpu.sync_copy(tok_hbm.at[wid], idx_v)

    def g_start(c, b):
        pltpu.async_copy(tab_hbm.at[idx_v.at[c]], rows[b], gsem[b])

    def g_wait(c, b):
        pltpu.make_async_copy(tab_hbm.at[idx_v.at[c]], rows[b], gsem[b]).wait()

    def o1_start(b):
        pltpu.async_copy(rows[b], sp[b], s1[b])

    def o1_wait(b):
        pltpu.make_async_copy(rows[b], sp[b], s1[b]).wait()

    def o2_start(c, b):
        pltpu.async_copy(sp[b], out_hbm.at[pl.ds(base + c * CH, CH)], s2[b])

    def o2_wait(c, b):
        pltpu.make_async_copy(
            sp[b], out_hbm.at[pl.ds(base + c * CH, CH)], s2[b]).wait()

    for c in range(AHEAD):
        g_start(c, c)

    def outer(i, carry):
        j = i * NBUF
        for b in range(NBUF):
            cur = j + b

            @pl.when(cur >= 1)
            def _():
                o1_wait((b - 1) % NBUF)
                o2_start(cur - 1, (b - 1) % NBUF)

            @pl.when(cur >= AHEAD)
            def _():
                o2_wait(cur - AHEAD, (b + AHEAD) % NBUF)

            @pl.when(cur + AHEAD < NCHUNK)
            def _():
                g_start(cur + AHEAD, (b + AHEAD) % NBUF)

            g_wait(cur, b)

            buf = rows[b]

            @plsc.parallel_loop(0, CH, unroll=4)
            def _(r):
                for c in range(EMBED // 16):
                    buf[r, pl.ds(c * 16, 16)] = buf[r, pl.ds(c * 16, 16)] * SCALE

            o1_start(b)
        return carry

    lax.fori_loop(0, NCHUNK // NBUF, outer, 0)
    o1_wait((NCHUNK - 1) % NBUF)
    o2_start(NCHUNK - 1, (NCHUNK - 1) % NBUF)
    for c in range(NCHUNK - AHEAD, NCHUNK):
        o2_wait(c, c % NBUF)


def kernel(tokens, table):
    tok = tokens.reshape(NW, NCHUNK, CH).astype(jnp.int32)
    out = _embed_sc(tok, table)
    return out.reshape(tokens.shape[0], tokens.shape[1], EMBED)
